# Initial kernel scaffold; baseline (speedup 1.0000x reference)
import jax
import jax.numpy as jnp
from jax.experimental import pallas as pl


def kernel(x, edge_index, W1, b1, W2, b2, Wfc, bfc):
    raise NotImplementedError("write your pallas kernel here")


# speedup vs baseline: 8.9974x; 8.9974x over previous
import functools

import jax
import jax.numpy as jnp
from jax import lax
from jax.experimental import pallas as pl
from jax.experimental.pallas import tpu as pltpu
from jax.experimental.pallas import tpu_sc as plsc

N = 10000
E = 320000
D = 128
NC = 2
NS = 16
CH = 128
CPT = 80
EP = NC * NS * CPT * CH
NP = 10240
RPT = NP // NS
NCHUNK = EP // CH
DEGW = 16

def _vmesh():
    return plsc.VectorSubcoreMesh(core_axis_name="c", subcore_axis_name="s")


@jax.jit
def _sc_degree(dst2d):
    @functools.partial(
        pl.kernel,
        out_type=jax.ShapeDtypeStruct((NC, NP, DEGW), jnp.float32),
        mesh=_vmesh(),
        scratch_types=[
            pltpu.VMEM((CPT, CH), jnp.int32),
            pltpu.VMEM((CH, DEGW), jnp.float32),
            pltpu.VMEM((CH, DEGW), jnp.float32),
            pltpu.VMEM_SHARED((NP, DEGW), jnp.float32),
        ],
    )
    def deg_kernel(dst_hbm, out_hbm, idx_v, ones_v, zero_v, acc):
        c = lax.axis_index("c")
        s = lax.axis_index("s")
        pltpu.sync_copy(dst_hbm.at[pl.ds((c * NS + s) * CPT, CPT)], idx_v)

        lane = lax.iota(jnp.int32, 16)
        onerow = jnp.where(lane == 0, 1.0, 0.0)
        zrow = jnp.zeros((16,), jnp.float32)

        @pl.loop(0, CH)
        def _fill(i):
            ones_v[i, :] = onerow
            zero_v[i, :] = zrow

        @pl.loop(0, RPT, step=CH)
        def _zero(r):
            pltpu.sync_copy(zero_v, acc.at[pl.ds(s * RPT + r, CH)])

        plsc.subcore_barrier()

        @pl.loop(0, CPT)
        def _scat(j):
            pltpu.sync_copy(ones_v, acc.at[idx_v.at[j]], add=True)

        plsc.subcore_barrier()
        pltpu.sync_copy(
            acc.at[pl.ds(s * RPT, RPT)],
            out_hbm.at[c, pl.ds(s * RPT, RPT)],
        )

    return deg_kernel(dst2d)


@jax.jit
def _sc_aggregate(g, src2d, dst2d):
    HPC = CPT // 2

    @functools.partial(
        pl.kernel,
        out_type=jax.ShapeDtypeStruct((NC, NP, D), jnp.float32),
        mesh=_vmesh(),
        scratch_types=[
            pltpu.VMEM((HPC, CH), jnp.int32),
            pltpu.VMEM((HPC, CH), jnp.int32),
            pltpu.VMEM((CH, D), jnp.float32),
            pltpu.VMEM((CH, D), jnp.float32),
            pltpu.VMEM_SHARED((NP, D), jnp.float32),
            pltpu.SemaphoreType.DMA,
            pltpu.SemaphoreType.DMA,
        ],
    )
    def agg_kernel(g_hbm, src_hbm, dst_hbm, out_hbm,
                   src_v, dst_v, rows0, rows1, acc, sem0, sem1):
        c = lax.axis_index("c")
        s = lax.axis_index("s")

        zrow = jnp.zeros((16,), jnp.float32)

        @pl.loop(0, CH)
        def _fill(i):
            @pl.loop(0, D, step=16)
            def _fill2(q):
                rows0[i, pl.ds(q, 16)] = zrow

        @pl.loop(0, RPT, step=CH)
        def _zero(r):
            pltpu.sync_copy(rows0, acc.at[pl.ds(s * RPT + r, CH)])

        plsc.subcore_barrier()

        rows = (rows0, rows1)
        sems = (sem0, sem1)

        for ph in range(2):
            base = (c * NS + s) * CPT + ph * HPC
            pltpu.sync_copy(src_hbm.at[pl.ds(base, HPC)], src_v)
            pltpu.sync_copy(dst_hbm.at[pl.ds(base, HPC)], dst_v)

            for b in range(2):
                pltpu.async_copy(g_hbm.at[src_v.at[b]], rows[b], sems[b])

            @pl.loop(0, HPC, step=2)
            def _edges(j):
                for b in range(2):
                    jb = j + b
                    pltpu.make_async_copy(
                        g_hbm.at[src_v.at[jb]], rows[b], sems[b]).wait()
                    pltpu.sync_copy(rows[b], acc.at[dst_v.at[jb]], add=True)

                    @pl.when(jb + 2 < HPC)
                    def _next():
                        pltpu.async_copy(
                            g_hbm.at[src_v.at[jb + 2]], rows[b], sems[b])

        plsc.subcore_barrier()
        pltpu.sync_copy(
            acc.at[pl.ds(s * RPT, RPT)],
            out_hbm.at[c, pl.ds(s * RPT, RPT)],
        )

    return agg_kernel(g, src2d, dst2d)


_BT = 1024


def _dinv_block(p0, p1):
    deg = 1.0 + p0[:, 0:1] + p1[:, 0:1]
    return lax.rsqrt(deg)


def _stage1_body(x_ref, w_ref, p0_ref, p1_ref, g_ref):
    dinv = _dinv_block(p0_ref[...], p1_ref[...])
    h = jnp.dot(x_ref[...], w_ref[...], preferred_element_type=jnp.float32)
    g_ref[...] = h * dinv


def _stage2_body(s0_ref, s1_ref, g_ref, p0_ref, p1_ref, b_ref, w_ref, o_ref):
    dinv = _dinv_block(p0_ref[...], p1_ref[...])
    h = dinv * (s0_ref[...] + s1_ref[...] + g_ref[...]) + b_ref[...]
    h = jnp.maximum(h, 0.0)
    o_ref[...] = jnp.dot(h, w_ref[...], preferred_element_type=jnp.float32) * dinv


def _stage3_body(s0_ref, s1_ref, g_ref, p0_ref, p1_ref, b_ref, w_ref, bf_ref, o_ref):
    dinv = _dinv_block(p0_ref[...], p1_ref[...])
    h = dinv * (s0_ref[...] + s1_ref[...] + g_ref[...]) + b_ref[...]
    h = jnp.maximum(h, 0.0)
    o_ref[...] = jnp.dot(h, w_ref[...], preferred_element_type=jnp.float32) + bf_ref[...]


_row_spec = pl.BlockSpec((_BT, D), lambda i: (i, 0))
_p_spec = pl.BlockSpec((_BT, DEGW), lambda i: (i, 0))
_w_spec = pl.BlockSpec((D, D), lambda i: (0, 0))
_b_spec = pl.BlockSpec((1, D), lambda i: (0, 0))
_out_struct = jax.ShapeDtypeStruct((NP, D), jnp.float32)
_grid = (NP // _BT,)


@jax.jit
def _tc_stage1(x, w1, p0, p1):
    return pl.pallas_call(
        _stage1_body,
        grid=_grid,
        in_specs=[_row_spec, _w_spec, _p_spec, _p_spec],
        out_specs=_row_spec,
        out_shape=_out_struct,
    )(x, w1, p0, p1)


@jax.jit
def _tc_stage2(s0, s1, g, p0, p1, b, w):
    return pl.pallas_call(
        _stage2_body,
        grid=_grid,
        in_specs=[_row_spec, _row_spec, _row_spec, _p_spec, _p_spec, _b_spec, _w_spec],
        out_specs=_row_spec,
        out_shape=_out_struct,
    )(s0, s1, g, p0, p1, b, w)


@jax.jit
def _tc_stage3(s0, s1, g, p0, p1, b, w, bf):
    return pl.pallas_call(
        _stage3_body,
        grid=_grid,
        in_specs=[_row_spec, _row_spec, _row_spec, _p_spec, _p_spec, _b_spec,
                  _w_spec, _b_spec],
        out_specs=_row_spec,
        out_shape=_out_struct,
    )(s0, s1, g, p0, p1, b, w, bf)


def kernel(x, edge_index, W1, b1, W2, b2, Wfc, bfc):
    src = edge_index[0]
    dst = edge_index[1]
    pad = jnp.full((EP - E,), N, jnp.int32)
    src2d = jnp.concatenate([src, pad]).reshape(NCHUNK, CH)
    dst2d = jnp.concatenate([dst, pad]).reshape(NCHUNK, CH)
    x_p = jnp.concatenate([x, jnp.zeros((NP - N, D), x.dtype)], axis=0)

    w_fc = jnp.zeros((D, D), jnp.float32).at[:, : Wfc.shape[1]].set(Wfc)
    b_fc = jnp.zeros((1, D), jnp.float32).at[0, : bfc.shape[0]].set(bfc)
    b1r = b1.reshape(1, D)
    b2r = b2.reshape(1, D)

    degp = _sc_degree(dst2d)
    p0, p1 = degp[0], degp[1]

    g1 = _tc_stage1(x_p, W1, p0, p1)
    s1 = _sc_aggregate(g1, src2d, dst2d)
    g2 = _tc_stage2(s1[0], s1[1], g1, p0, p1, b1r, W2)
    s2 = _sc_aggregate(g2, src2d, dst2d)
    out = _tc_stage3(s2[0], s2[1], g2, p0, p1, b2r, w_fc, b_fc)
    return out[:N, : Wfc.shape[1]]
